# baseline (speedup 1.0000x reference)
import functools

import jax
import jax.numpy as jnp
from jax import lax
from jax.experimental import pallas as pl
from jax.experimental.pallas import tpu as pltpu
from jax.experimental.pallas import tpu_sc as plsc

T = 2048
D = 1024
E = 8
FFN = 512
TOPK = 2
NGROUP = 4
EPG = E // NGROUP
SFFN = 512
SCALE = 2.5

TM = 256
NT = T // TM

NC = 2
NS = 16
NW = NC * NS
TW = T // NW
L = 16


def _silu(x):
    return x * jax.nn.sigmoid(x)


def _sel_body(scores_hbm, sb_hbm, gsum_hbm, out_hbm, sc_v, sb_v, gs_v, cb_v):
    wid = lax.axis_index("s") * NC + lax.axis_index("c")
    pltpu.sync_copy(scores_hbm.at[wid], sc_v)
    pltpu.sync_copy(sb_hbm.at[wid], sb_v)
    pltpu.sync_copy(gsum_hbm.at[wid], gs_v)

    neg = jnp.float32(-jnp.inf)
    for c in range(TW // L):
        sl = pl.ds(c * L, L)
        s = [sc_v[e, sl] for e in range(E)]
        sb = [sb_v[e, sl] for e in range(E)]
        gs = [gs_v[g, sl] for g in range(NGROUP)]

        m1 = jnp.maximum(jnp.maximum(gs[0], gs[1]), jnp.maximum(gs[2], gs[3]))
        g1 = jnp.where(gs[0] == m1, 0,
             jnp.where(gs[1] == m1, 1,
             jnp.where(gs[2] == m1, 2, 3))).astype(jnp.int32)
        gs2 = [jnp.where(g1 == g, neg, gs[g]) for g in range(NGROUP)]
        m2 = jnp.maximum(jnp.maximum(gs2[0], gs2[1]),
                         jnp.maximum(gs2[2], gs2[3]))
        g2 = jnp.where(gs2[0] == m2, 0,
             jnp.where(gs2[1] == m2, 1,
             jnp.where(gs2[2] == m2, 2, 3))).astype(jnp.int32)

        tmp = [jnp.where((g1 == (e // EPG)) | (g2 == (e // EPG)), sb[e],
                         jnp.float32(0.0)) for e in range(E)]
        t1 = tmp[0]
        for e in range(1, E):
            t1 = jnp.maximum(t1, tmp[e])
        e1 = jnp.full((L,), E - 1, jnp.int32)
        for e in range(E - 2, -1, -1):
            e1 = jnp.where(tmp[e] == t1, e, e1)
        tmp2 = [jnp.where(e1 == e, neg, tmp[e]) for e in range(E)]
        t2 = tmp2[0]
        for e in range(1, E):
            t2 = jnp.maximum(t2, tmp2[e])
        e2 = jnp.full((L,), E - 1, jnp.int32)
        for e in range(E - 2, -1, -1):
            e2 = jnp.where(tmp2[e] == t2, e, e2)

        w1 = s[E - 1]
        w2 = s[E - 1]
        for e in range(E - 2, -1, -1):
            w1 = jnp.where(e1 == e, s[e], w1)
            w2 = jnp.where(e2 == e, s[e], w2)
        denom = w1 + w2
        c1 = w1 / denom
        c2 = w2 / denom
        for e in range(E):
            cb_v[e, sl] = jnp.where(e1 == e, c1,
                          jnp.where(e2 == e, c2, jnp.float32(0.0)))

    pltpu.sync_copy(cb_v, out_hbm.at[wid])


def _route_sc(scores_r, sb_r, gsum_r):
    mesh = plsc.VectorSubcoreMesh(core_axis_name="c", subcore_axis_name="s")
    fn = functools.partial(
        pl.kernel,
        mesh=mesh,
        out_type=jax.ShapeDtypeStruct((NW, E, TW), jnp.float32),
        scratch_types=[
            pltpu.VMEM((E, TW), jnp.float32),
            pltpu.VMEM((E, TW), jnp.float32),
            pltpu.VMEM((NGROUP, TW), jnp.float32),
            pltpu.VMEM((E, TW), jnp.float32),
        ],
    )(_sel_body)
    return fn(scores_r, sb_r, gsum_r)


def _moe_body(x_ref, comb_ref, wgu_ref, wd_ref, sgu_ref, sd_ref, out_ref):
    xs = x_ref[...]

    cs = SCALE * comb_ref[...]

    gu = lax.dot_general(xs, sgu_ref[...], (((1,), (0,)), ((), ())),
                         preferred_element_type=jnp.float32)
    h = (_silu(gu[:, :SFFN]) * gu[:, SFFN:]).astype(jnp.bfloat16)
    acc = lax.dot_general(h, sd_ref[...], (((1,), (0,)), ((), ())),
                          preferred_element_type=jnp.float32)

    for e in range(E):
        gue = lax.dot_general(xs, wgu_ref[e], (((1,), (0,)), ((), ())),
                              preferred_element_type=jnp.float32)
        he = (_silu(gue[:, :FFN]) * gue[:, FFN:]).astype(jnp.bfloat16)
        ye = lax.dot_general(he, wd_ref[e], (((1,), (0,)), ((), ())),
                             preferred_element_type=jnp.float32)
        acc = acc + cs[:, e:e + 1] * ye

    out_ref[...] = acc


def _moe(x_bf, comb, wgu_t, wd_t, sgu_t, sd_t):
    return pl.pallas_call(
        _moe_body,
        grid=(NT,),
        in_specs=[
            pl.BlockSpec((TM, D), lambda t: (t, 0)),
            pl.BlockSpec((TM, E), lambda t: (t, 0)),
            pl.BlockSpec((E, D, 2 * FFN), lambda t: (0, 0, 0)),
            pl.BlockSpec((E, FFN, D), lambda t: (0, 0, 0)),
            pl.BlockSpec((D, 2 * SFFN), lambda t: (0, 0)),
            pl.BlockSpec((SFFN, D), lambda t: (0, 0)),
        ],
        out_specs=pl.BlockSpec((TM, D), lambda t: (t, 0)),
        out_shape=jax.ShapeDtypeStruct((T, D), jnp.float32),
        compiler_params=pltpu.CompilerParams(
            vmem_limit_bytes=60 * 1024 * 1024),
    )(x_bf, comb, wgu_t, wd_t, sgu_t, sd_t)


@jax.jit
def _glm4_moe(hidden_states, gate_w, corr_bias, w_gate_up, w_down, s_gate_up,
              s_down):
    router_logits = hidden_states.astype(jnp.float32) @ gate_w.T
    scores = jax.nn.sigmoid(router_logits)
    sb = scores + corr_bias[None, :]
    sba, sbb = sb[:, 0::2], sb[:, 1::2]
    gsum = jnp.maximum(sba, sbb) + jnp.minimum(sba, sbb)

    def to_r(a, rows):
        return a.T.reshape(rows, NW, TW).transpose(1, 0, 2)

    comb3 = _route_sc(to_r(scores, E), to_r(sb, E), to_r(gsum, NGROUP))
    comb = comb3.transpose(0, 2, 1).reshape(T, E)

    bf = jnp.bfloat16
    x_bf = hidden_states.astype(bf)
    wgu_t = w_gate_up.astype(bf).transpose(0, 2, 1)
    wd_t = w_down.astype(bf).transpose(0, 2, 1)
    sgu_t = s_gate_up.astype(bf).T
    sd_t = s_down.astype(bf).T
    return _moe(x_bf, comb, wgu_t, wd_t, sgu_t, sd_t)


def kernel(hidden_states, gate_w, corr_bias, w_gate_up, w_down, s_gate_up,
           s_down):
    return _glm4_moe(hidden_states, gate_w, corr_bias, w_gate_up, w_down,
                     s_gate_up, s_down)


# speedup vs baseline: 1.0762x; 1.0762x over previous
import functools

import jax
import jax.numpy as jnp
from jax import lax
from jax.experimental import pallas as pl
from jax.experimental.pallas import tpu as pltpu
from jax.experimental.pallas import tpu_sc as plsc

T = 2048
D = 1024
E = 8
FFN = 512
TOPK = 2
NGROUP = 4
EPG = E // NGROUP
SFFN = 512
SCALE = 2.5

TM = 512
NT = T // TM

NC = 2
NS = 16
NW = NC * NS
TW = T // NW
L = 16


def _silu(x):
    return x * jax.nn.sigmoid(x)


def _sel_body(scores_hbm, sb_hbm, gsum_hbm, out_hbm, sc_v, sb_v, gs_v, cb_v):
    wid = lax.axis_index("s") * NC + lax.axis_index("c")
    pltpu.sync_copy(scores_hbm.at[wid], sc_v)
    pltpu.sync_copy(sb_hbm.at[wid], sb_v)
    pltpu.sync_copy(gsum_hbm.at[wid], gs_v)

    neg = jnp.float32(-jnp.inf)
    for c in range(TW // L):
        sl = pl.ds(c * L, L)
        s = [sc_v[e, sl] for e in range(E)]
        sb = [sb_v[e, sl] for e in range(E)]
        gs = [gs_v[g, sl] for g in range(NGROUP)]

        m1 = jnp.maximum(jnp.maximum(gs[0], gs[1]), jnp.maximum(gs[2], gs[3]))
        g1 = jnp.where(gs[0] == m1, 0,
             jnp.where(gs[1] == m1, 1,
             jnp.where(gs[2] == m1, 2, 3))).astype(jnp.int32)
        gs2 = [jnp.where(g1 == g, neg, gs[g]) for g in range(NGROUP)]
        m2 = jnp.maximum(jnp.maximum(gs2[0], gs2[1]),
                         jnp.maximum(gs2[2], gs2[3]))
        g2 = jnp.where(gs2[0] == m2, 0,
             jnp.where(gs2[1] == m2, 1,
             jnp.where(gs2[2] == m2, 2, 3))).astype(jnp.int32)

        tmp = [jnp.where((g1 == (e // EPG)) | (g2 == (e // EPG)), sb[e],
                         jnp.float32(0.0)) for e in range(E)]
        t1 = tmp[0]
        for e in range(1, E):
            t1 = jnp.maximum(t1, tmp[e])
        e1 = jnp.full((L,), E - 1, jnp.int32)
        for e in range(E - 2, -1, -1):
            e1 = jnp.where(tmp[e] == t1, e, e1)
        tmp2 = [jnp.where(e1 == e, neg, tmp[e]) for e in range(E)]
        t2 = tmp2[0]
        for e in range(1, E):
            t2 = jnp.maximum(t2, tmp2[e])
        e2 = jnp.full((L,), E - 1, jnp.int32)
        for e in range(E - 2, -1, -1):
            e2 = jnp.where(tmp2[e] == t2, e, e2)

        w1 = s[E - 1]
        w2 = s[E - 1]
        for e in range(E - 2, -1, -1):
            w1 = jnp.where(e1 == e, s[e], w1)
            w2 = jnp.where(e2 == e, s[e], w2)
        denom = w1 + w2
        c1 = w1 / denom
        c2 = w2 / denom
        for e in range(E):
            cb_v[e, sl] = jnp.where(e1 == e, c1,
                          jnp.where(e2 == e, c2, jnp.float32(0.0)))

    pltpu.sync_copy(cb_v, out_hbm.at[wid])


def _route_sc(scores_r, sb_r, gsum_r):
    mesh = plsc.VectorSubcoreMesh(core_axis_name="c", subcore_axis_name="s")
    fn = functools.partial(
        pl.kernel,
        mesh=mesh,
        out_type=jax.ShapeDtypeStruct((NW, E, TW), jnp.float32),
        scratch_types=[
            pltpu.VMEM((E, TW), jnp.float32),
            pltpu.VMEM((E, TW), jnp.float32),
            pltpu.VMEM((NGROUP, TW), jnp.float32),
            pltpu.VMEM((E, TW), jnp.float32),
        ],
    )(_sel_body)
    return fn(scores_r, sb_r, gsum_r)


def _moe_body(x_ref, comb_ref, wgu_ref, wd_ref, sgu_ref, sd_ref, out_ref):
    xs = x_ref[...].astype(jnp.bfloat16)

    cs = SCALE * comb_ref[...]

    gu = lax.dot_general(xs, sgu_ref[...], (((1,), (0,)), ((), ())),
                         preferred_element_type=jnp.float32)
    h = (_silu(gu[:, :SFFN]) * gu[:, SFFN:]).astype(jnp.bfloat16)
    acc = lax.dot_general(h, sd_ref[...], (((1,), (0,)), ((), ())),
                          preferred_element_type=jnp.float32)

    for e in range(E):
        gue = lax.dot_general(xs, wgu_ref[e], (((1,), (0,)), ((), ())),
                              preferred_element_type=jnp.float32)
        he = (_silu(gue[:, :FFN]) * gue[:, FFN:]).astype(jnp.bfloat16)
        ye = lax.dot_general(he, wd_ref[e], (((1,), (0,)), ((), ())),
                             preferred_element_type=jnp.float32)
        acc = acc + cs[:, e:e + 1] * ye

    out_ref[...] = acc


def _moe(x_bf, comb, wgu_t, wd_t, sgu_t, sd_t):
    return pl.pallas_call(
        _moe_body,
        grid=(NT,),
        in_specs=[
            pl.BlockSpec((TM, D), lambda t: (t, 0)),
            pl.BlockSpec((TM, E), lambda t: (t, 0)),
            pl.BlockSpec((E, D, 2 * FFN), lambda t: (0, 0, 0)),
            pl.BlockSpec((E, FFN, D), lambda t: (0, 0, 0)),
            pl.BlockSpec((D, 2 * SFFN), lambda t: (0, 0)),
            pl.BlockSpec((SFFN, D), lambda t: (0, 0)),
        ],
        out_specs=pl.BlockSpec((TM, D), lambda t: (t, 0)),
        out_shape=jax.ShapeDtypeStruct((T, D), jnp.float32),
        compiler_params=pltpu.CompilerParams(
            vmem_limit_bytes=60 * 1024 * 1024),
    )(x_bf, comb, wgu_t, wd_t, sgu_t, sd_t)


@jax.jit
def _glm4_moe(hidden_states, gate_w, corr_bias, w_gate_up, w_down, s_gate_up,
              s_down):
    router_logits = hidden_states.astype(jnp.float32) @ gate_w.T
    scores = jax.nn.sigmoid(router_logits)
    sb = scores + corr_bias[None, :]
    sba, sbb = sb[:, 0::2], sb[:, 1::2]
    gsum = jnp.maximum(sba, sbb) + jnp.minimum(sba, sbb)

    def to_r(a, rows):
        return a.T.reshape(rows, NW, TW).transpose(1, 0, 2)

    comb3 = _route_sc(to_r(scores, E), to_r(sb, E), to_r(gsum, NGROUP))
    comb = comb3.transpose(0, 2, 1).reshape(T, E)

    bf = jnp.bfloat16
    wgu_t = w_gate_up.astype(bf).transpose(0, 2, 1)
    wd_t = w_down.astype(bf).transpose(0, 2, 1)
    sgu_t = s_gate_up.astype(bf).T
    sd_t = s_down.astype(bf).T
    return _moe(hidden_states, comb, wgu_t, wd_t, sgu_t, sd_t)


def kernel(hidden_states, gate_w, corr_bias, w_gate_up, w_down, s_gate_up,
           s_down):
    return _glm4_moe(hidden_states, gate_w, corr_bias, w_gate_up, w_down,
                     s_gate_up, s_down)


# speedup vs baseline: 1.2172x; 1.1310x over previous
import functools

import jax
import jax.numpy as jnp
from jax import lax
from jax.experimental import pallas as pl
from jax.experimental.pallas import tpu as pltpu
from jax.experimental.pallas import tpu_sc as plsc

T = 2048
D = 1024
E = 8
FFN = 512
TOPK = 2
NGROUP = 4
EPG = E // NGROUP
SFFN = 512
SCALE = 2.5

TM = 512
NT = T // TM

NC = 2
NS = 16
NW = NC * NS
TW = T // NW
L = 16


def _silu(x):
    return x * jax.nn.sigmoid(x)


def _sel_body(scores_hbm, sb_hbm, gsum_hbm, out_hbm, sc_v, sb_v, gs_v, cb_v):
    wid = lax.axis_index("s") * NC + lax.axis_index("c")
    pltpu.sync_copy(scores_hbm.at[wid], sc_v)
    pltpu.sync_copy(sb_hbm.at[wid], sb_v)
    pltpu.sync_copy(gsum_hbm.at[wid], gs_v)

    neg = jnp.float32(-jnp.inf)
    for c in range(TW // L):
        sl = pl.ds(c * L, L)
        s = [sc_v[e, sl] for e in range(E)]
        sb = [sb_v[e, sl] for e in range(E)]
        gs = [gs_v[g, sl] for g in range(NGROUP)]

        m1 = jnp.maximum(jnp.maximum(gs[0], gs[1]), jnp.maximum(gs[2], gs[3]))
        g1 = jnp.where(gs[0] == m1, 0,
             jnp.where(gs[1] == m1, 1,
             jnp.where(gs[2] == m1, 2, 3))).astype(jnp.int32)
        gs2 = [jnp.where(g1 == g, neg, gs[g]) for g in range(NGROUP)]
        m2 = jnp.maximum(jnp.maximum(gs2[0], gs2[1]),
                         jnp.maximum(gs2[2], gs2[3]))
        g2 = jnp.where(gs2[0] == m2, 0,
             jnp.where(gs2[1] == m2, 1,
             jnp.where(gs2[2] == m2, 2, 3))).astype(jnp.int32)

        tmp = [jnp.where((g1 == (e // EPG)) | (g2 == (e // EPG)), sb[e],
                         jnp.float32(0.0)) for e in range(E)]
        t1 = tmp[0]
        for e in range(1, E):
            t1 = jnp.maximum(t1, tmp[e])
        e1 = jnp.full((L,), E - 1, jnp.int32)
        for e in range(E - 2, -1, -1):
            e1 = jnp.where(tmp[e] == t1, e, e1)
        tmp2 = [jnp.where(e1 == e, neg, tmp[e]) for e in range(E)]
        t2 = tmp2[0]
        for e in range(1, E):
            t2 = jnp.maximum(t2, tmp2[e])
        e2 = jnp.full((L,), E - 1, jnp.int32)
        for e in range(E - 2, -1, -1):
            e2 = jnp.where(tmp2[e] == t2, e, e2)

        w1 = s[E - 1]
        w2 = s[E - 1]
        for e in range(E - 2, -1, -1):
            w1 = jnp.where(e1 == e, s[e], w1)
            w2 = jnp.where(e2 == e, s[e], w2)
        denom = w1 + w2
        c1 = w1 / denom
        c2 = w2 / denom
        for e in range(E):
            cb_v[e, sl] = jnp.where(e1 == e, c1,
                          jnp.where(e2 == e, c2, jnp.float32(0.0)))

    pltpu.sync_copy(cb_v, out_hbm.at[wid])


def _route_sc(scores_r, sb_r, gsum_r):
    mesh = plsc.VectorSubcoreMesh(core_axis_name="c", subcore_axis_name="s")
    fn = functools.partial(
        pl.kernel,
        mesh=mesh,
        out_type=jax.ShapeDtypeStruct((NW, E, TW), jnp.float32),
        scratch_types=[
            pltpu.VMEM((E, TW), jnp.float32),
            pltpu.VMEM((E, TW), jnp.float32),
            pltpu.VMEM((NGROUP, TW), jnp.float32),
            pltpu.VMEM((E, TW), jnp.float32),
        ],
    )(_sel_body)
    return fn(scores_r, sb_r, gsum_r)


def _moe_body(x_ref, comb_ref, wgu_ref, wd_ref, sgu_ref, sd_ref, out_ref):
    xs = x_ref[...].astype(jnp.bfloat16)

    cs = SCALE * comb_ref[...]

    gu = lax.dot_general(xs, sgu_ref[...], (((1,), (1,)), ((), ())),
                         preferred_element_type=jnp.float32)
    h = (_silu(gu[:, :SFFN]) * gu[:, SFFN:]).astype(jnp.bfloat16)
    acc = lax.dot_general(h, sd_ref[...], (((1,), (1,)), ((), ())),
                          preferred_element_type=jnp.float32)

    for e in range(E):
        gue = lax.dot_general(xs, wgu_ref[e], (((1,), (1,)), ((), ())),
                              preferred_element_type=jnp.float32)
        he = (_silu(gue[:, :FFN]) * gue[:, FFN:]).astype(jnp.bfloat16)
        ye = lax.dot_general(he, wd_ref[e], (((1,), (1,)), ((), ())),
                             preferred_element_type=jnp.float32)
        acc = acc + cs[:, e:e + 1] * ye

    out_ref[...] = acc


def _moe(x_bf, comb, wgu_t, wd_t, sgu_t, sd_t):
    return pl.pallas_call(
        _moe_body,
        grid=(NT,),
        in_specs=[
            pl.BlockSpec((TM, D), lambda t: (t, 0)),
            pl.BlockSpec((TM, E), lambda t: (t, 0)),
            pl.BlockSpec((E, 2 * FFN, D), lambda t: (0, 0, 0)),
            pl.BlockSpec((E, D, FFN), lambda t: (0, 0, 0)),
            pl.BlockSpec((2 * SFFN, D), lambda t: (0, 0)),
            pl.BlockSpec((D, SFFN), lambda t: (0, 0)),
        ],
        out_specs=pl.BlockSpec((TM, D), lambda t: (t, 0)),
        out_shape=jax.ShapeDtypeStruct((T, D), jnp.float32),
        compiler_params=pltpu.CompilerParams(
            vmem_limit_bytes=60 * 1024 * 1024),
    )(x_bf, comb, wgu_t, wd_t, sgu_t, sd_t)


@jax.jit
def _glm4_moe(hidden_states, gate_w, corr_bias, w_gate_up, w_down, s_gate_up,
              s_down):
    router_logits = hidden_states.astype(jnp.float32) @ gate_w.T
    scores = jax.nn.sigmoid(router_logits)
    sb = scores + corr_bias[None, :]
    sba, sbb = sb[:, 0::2], sb[:, 1::2]
    gsum = jnp.maximum(sba, sbb) + jnp.minimum(sba, sbb)

    def to_r(a, rows):
        return a.T.reshape(rows, NW, TW).transpose(1, 0, 2)

    comb3 = _route_sc(to_r(scores, E), to_r(sb, E), to_r(gsum, NGROUP))
    comb = comb3.transpose(0, 2, 1).reshape(T, E)

    bf = jnp.bfloat16
    wgu_t = w_gate_up.astype(bf)
    wd_t = w_down.astype(bf)
    sgu_t = s_gate_up.astype(bf)
    sd_t = s_down.astype(bf)
    return _moe(hidden_states, comb, wgu_t, wd_t, sgu_t, sd_t)


def kernel(hidden_states, gate_w, corr_bias, w_gate_up, w_down, s_gate_up,
           s_down):
    return _glm4_moe(hidden_states, gate_w, corr_bias, w_gate_up, w_down,
                     s_gate_up, s_down)


# speedup vs baseline: 1.2455x; 1.0233x over previous
import functools

import jax
import jax.numpy as jnp
from jax import lax
from jax.experimental import pallas as pl
from jax.experimental.pallas import tpu as pltpu
from jax.experimental.pallas import tpu_sc as plsc

T = 2048
D = 1024
E = 8
FFN = 512
TOPK = 2
NGROUP = 4
EPG = E // NGROUP
SFFN = 512
SCALE = 2.5

TM = 1024
NT = T // TM

NC = 2
NS = 16
NW = NC * NS
TW = T // NW
L = 16


def _silu(x):
    return x * jax.nn.sigmoid(x)


def _sel_body(scores_hbm, sb_hbm, gsum_hbm, out_hbm, sc_v, sb_v, gs_v, cb_v):
    wid = lax.axis_index("s") * NC + lax.axis_index("c")
    pltpu.sync_copy(scores_hbm.at[wid], sc_v)
    pltpu.sync_copy(sb_hbm.at[wid], sb_v)
    pltpu.sync_copy(gsum_hbm.at[wid], gs_v)

    neg = jnp.float32(-jnp.inf)
    for c in range(TW // L):
        sl = pl.ds(c * L, L)
        s = [sc_v[e, sl] for e in range(E)]
        sb = [sb_v[e, sl] for e in range(E)]
        gs = [gs_v[g, sl] for g in range(NGROUP)]

        m1 = jnp.maximum(jnp.maximum(gs[0], gs[1]), jnp.maximum(gs[2], gs[3]))
        g1 = jnp.where(gs[0] == m1, 0,
             jnp.where(gs[1] == m1, 1,
             jnp.where(gs[2] == m1, 2, 3))).astype(jnp.int32)
        gs2 = [jnp.where(g1 == g, neg, gs[g]) for g in range(NGROUP)]
        m2 = jnp.maximum(jnp.maximum(gs2[0], gs2[1]),
                         jnp.maximum(gs2[2], gs2[3]))
        g2 = jnp.where(gs2[0] == m2, 0,
             jnp.where(gs2[1] == m2, 1,
             jnp.where(gs2[2] == m2, 2, 3))).astype(jnp.int32)

        tmp = [jnp.where((g1 == (e // EPG)) | (g2 == (e // EPG)), sb[e],
                         jnp.float32(0.0)) for e in range(E)]
        t1 = tmp[0]
        for e in range(1, E):
            t1 = jnp.maximum(t1, tmp[e])
        e1 = jnp.full((L,), E - 1, jnp.int32)
        for e in range(E - 2, -1, -1):
            e1 = jnp.where(tmp[e] == t1, e, e1)
        tmp2 = [jnp.where(e1 == e, neg, tmp[e]) for e in range(E)]
        t2 = tmp2[0]
        for e in range(1, E):
            t2 = jnp.maximum(t2, tmp2[e])
        e2 = jnp.full((L,), E - 1, jnp.int32)
        for e in range(E - 2, -1, -1):
            e2 = jnp.where(tmp2[e] == t2, e, e2)

        w1 = s[E - 1]
        w2 = s[E - 1]
        for e in range(E - 2, -1, -1):
            w1 = jnp.where(e1 == e, s[e], w1)
            w2 = jnp.where(e2 == e, s[e], w2)
        denom = w1 + w2
        c1 = w1 / denom
        c2 = w2 / denom
        for e in range(E):
            cb_v[e, sl] = jnp.where(e1 == e, c1,
                          jnp.where(e2 == e, c2, jnp.float32(0.0)))

    pltpu.sync_copy(cb_v, out_hbm.at[wid])


def _route_sc(scores_r, sb_r, gsum_r):
    mesh = plsc.VectorSubcoreMesh(core_axis_name="c", subcore_axis_name="s")
    fn = functools.partial(
        pl.kernel,
        mesh=mesh,
        out_type=jax.ShapeDtypeStruct((NW, E, TW), jnp.float32),
        scratch_types=[
            pltpu.VMEM((E, TW), jnp.float32),
            pltpu.VMEM((E, TW), jnp.float32),
            pltpu.VMEM((NGROUP, TW), jnp.float32),
            pltpu.VMEM((E, TW), jnp.float32),
        ],
    )(_sel_body)
    return fn(scores_r, sb_r, gsum_r)


def _moe_body(x_ref, comb_ref, wgu_ref, wd_ref, sgu_ref, sd_ref, out_ref):
    xs = x_ref[...].astype(jnp.bfloat16)

    cs = SCALE * comb_ref[...]

    gu = lax.dot_general(xs, sgu_ref[...], (((1,), (1,)), ((), ())),
                         preferred_element_type=jnp.float32)
    h = (_silu(gu[:, :SFFN]) * gu[:, SFFN:]).astype(jnp.bfloat16)
    acc = lax.dot_general(h, sd_ref[...], (((1,), (1,)), ((), ())),
                          preferred_element_type=jnp.float32)

    for e in range(E):
        gue = lax.dot_general(xs, wgu_ref[e], (((1,), (1,)), ((), ())),
                              preferred_element_type=jnp.float32)
        he = (_silu(gue[:, :FFN]) * gue[:, FFN:]).astype(jnp.bfloat16)
        ye = lax.dot_general(he, wd_ref[e], (((1,), (1,)), ((), ())),
                             preferred_element_type=jnp.float32)
        acc = acc + cs[:, e:e + 1] * ye

    out_ref[...] = acc


def _moe(x_bf, comb, wgu_t, wd_t, sgu_t, sd_t):
    return pl.pallas_call(
        _moe_body,
        grid=(NT,),
        in_specs=[
            pl.BlockSpec((TM, D), lambda t: (t, 0)),
            pl.BlockSpec((TM, E), lambda t: (t, 0)),
            pl.BlockSpec((E, 2 * FFN, D), lambda t: (0, 0, 0)),
            pl.BlockSpec((E, D, FFN), lambda t: (0, 0, 0)),
            pl.BlockSpec((2 * SFFN, D), lambda t: (0, 0)),
            pl.BlockSpec((D, SFFN), lambda t: (0, 0)),
        ],
        out_specs=pl.BlockSpec((TM, D), lambda t: (t, 0)),
        out_shape=jax.ShapeDtypeStruct((T, D), jnp.float32),
        compiler_params=pltpu.CompilerParams(
            vmem_limit_bytes=60 * 1024 * 1024),
    )(x_bf, comb, wgu_t, wd_t, sgu_t, sd_t)


@jax.jit
def _glm4_moe(hidden_states, gate_w, corr_bias, w_gate_up, w_down, s_gate_up,
              s_down):
    router_logits = hidden_states.astype(jnp.float32) @ gate_w.T
    scores = jax.nn.sigmoid(router_logits)
    sb = scores + corr_bias[None, :]
    sba, sbb = sb[:, 0::2], sb[:, 1::2]
    gsum = jnp.maximum(sba, sbb) + jnp.minimum(sba, sbb)

    def to_r(a, rows):
        return a.T.reshape(rows, NW, TW).transpose(1, 0, 2)

    comb3 = _route_sc(to_r(scores, E), to_r(sb, E), to_r(gsum, NGROUP))
    comb = comb3.transpose(0, 2, 1).reshape(T, E)

    bf = jnp.bfloat16
    wgu_t = w_gate_up.astype(bf)
    wd_t = w_down.astype(bf)
    sgu_t = s_gate_up.astype(bf)
    sd_t = s_down.astype(bf)
    return _moe(hidden_states, comb, wgu_t, wd_t, sgu_t, sd_t)


def kernel(hidden_states, gate_w, corr_bias, w_gate_up, w_down, s_gate_up,
           s_down):
    return _glm4_moe(hidden_states, gate_w, corr_bias, w_gate_up, w_down,
                     s_gate_up, s_down)
